# baseline (speedup 1.0000x reference)
import functools

import jax
import jax.numpy as jnp
from jax import lax
from jax.experimental import pallas as pl
from jax.experimental.pallas import tpu as pltpu
from jax.experimental.pallas import tpu_sc as plsc

_NC = 2
_NS = 16
_NW = _NC * _NS
_G = 128
_EPS = 1e-12


def _rsqrt_vec(v):
    bits = lax.bitcast_convert_type(v, jnp.int32)
    y = lax.bitcast_convert_type(
        jnp.int32(0x5F3759DF) - lax.shift_right_logical(bits, 1), jnp.float32)
    for _ in range(3):
        y = y * (1.5 - 0.5 * v * y * y)
    return y


def _make_fused(n, d, l, max_len, chunk):
    per_w = n // _NW
    n_chunks = per_w // chunk
    gpc = chunk // _G
    d2 = 2 * d
    mesh = plsc.VectorSubcoreMesh(core_axis_name="c", subcore_axis_name="s")

    @functools.partial(
        pl.kernel,
        mesh=mesh,
        out_type=jax.ShapeDtypeStruct((n, d), jnp.float32),
        scratch_types=[
            pltpu.VMEM((n // _NW // _G, _G), jnp.int32),
            pltpu.VMEM((chunk, d2), jnp.float32),
            pltpu.VMEM((chunk, d), jnp.float32),
            pltpu.VMEM((max_len, d2), jnp.float32),
            pltpu.VMEM((8, d2), jnp.float32),
            pltpu.VMEM((8, d2), jnp.float32),
            pltpu.SemaphoreType.DMA,
        ],
        compiler_params=pltpu.CompilerParams(needs_layout_passes=False),
    )
    def fused_k(idx_hbm, tab_hbm, pe_hbm, g_hbm, b_hbm, out_hbm,
                idx_v, rows_v, pack_v, pe_v, g_v, b_v, sem):
        wid = lax.axis_index("s") * _NC + lax.axis_index("c")
        base = wid * per_w
        idx_rows = per_w // _G
        pltpu.sync_copy(
            idx_hbm.at[pl.ds(pl.multiple_of(wid * idx_rows, 8), idx_rows)],
            idx_v)
        pltpu.sync_copy(pe_hbm, pe_v)
        pltpu.sync_copy(g_hbm, g_v)
        pltpu.sync_copy(b_hbm, b_v)
        gs = [g_v[0, pl.ds(k * 16, 16)] for k in range(d // 16)]
        bs = [b_v[0, pl.ds(k * 16, 16)] for k in range(d // 16)]

        def body(i, carry):
            r0 = pl.multiple_of(base + i * chunk, 256)
            hs = [
                pltpu.async_copy(tab_hbm.at[idx_v.at[i * gpc + g]],
                                 rows_v.at[pl.ds(g * _G, _G)], sem)
                for g in range(gpc)
            ]
            for h in hs:
                h.wait()

            def row(j, c2):
                p = lax.rem(r0 + j, l)
                xs = [rows_v[j, pl.ds(k * 16, 16)] + pe_v[p, pl.ds(k * 16, 16)]
                      for k in range(d // 16)]
                s = xs[0] + xs[1] + xs[2] + xs[3]
                q = xs[0] * xs[0] + xs[1] * xs[1] + xs[2] * xs[2] + xs[3] * xs[3]
                total = jnp.sum(s)
                sumsq = jnp.sum(q)
                mean = total * (1.0 / d)
                var = sumsq * (1.0 / d) - mean * mean
                vv = jnp.full((16,), var + _EPS, dtype=jnp.float32)
                rstd = _rsqrt_vec(vv)
                for k in range(d // 16):
                    pack_v[j, pl.ds(k * 16, 16)] = (
                        (xs[k] - mean) * rstd * gs[k] + bs[k])
                return c2

            lax.fori_loop(0, chunk, row, 0)
            pltpu.sync_copy(pack_v, out_hbm.at[pl.ds(r0, chunk)])
            return carry

        lax.fori_loop(0, n_chunks, body, 0)

    return fused_k


def kernel(input_ids, W_emb, pe, ln_gamma, ln_beta):
    b, l = input_ids.shape
    v, d = W_emb.shape
    n = b * l
    idx = input_ids.reshape(n // _G, _G)
    tab = jnp.pad(W_emb, ((0, 0), (0, d)))
    fused = _make_fused(n, d, l, pe.shape[0], 256)
    pe_pad = jnp.pad(pe, ((0, 0), (0, d)))
    g_pad = jnp.pad(ln_gamma.reshape(1, d), ((0, 7), (0, d)))
    b_pad = jnp.pad(ln_beta.reshape(1, d), ((0, 7), (0, d)))
    out = fused(idx, tab, pe_pad, g_pad, b_pad)
    return out.reshape(b, l, d)


# speedup vs baseline: 1.2326x; 1.2326x over previous
import functools

import jax
import jax.numpy as jnp
from jax import lax
from jax.experimental import pallas as pl
from jax.experimental.pallas import tpu as pltpu
from jax.experimental.pallas import tpu_sc as plsc

_NC = 2
_NS = 16
_NW = _NC * _NS
_G = 128
_U = 8
_EPS = 1e-12


def _rsqrt_vec(v):
    bits = lax.bitcast_convert_type(v, jnp.int32)
    y = lax.bitcast_convert_type(
        jnp.int32(0x5F3759DF) - lax.shift_right_logical(bits, 1), jnp.float32)
    for _ in range(2):
        y = y * (1.5 - 0.5 * v * y * y)
    return y


def _make_fused(n, d, l):
    per_w = n // _NW
    n_chunks = per_w // _G
    d2 = 2 * d
    nk = d // 16
    mesh = plsc.VectorSubcoreMesh(core_axis_name="c", subcore_axis_name="s")

    @functools.partial(
        pl.kernel,
        mesh=mesh,
        out_type=jax.ShapeDtypeStruct((n, d), jnp.float32),
        scratch_types=[
            pltpu.VMEM((n_chunks, _G), jnp.int32),
            pltpu.VMEM((2, _G, d2), jnp.float32),
            pltpu.VMEM((2, _G, d), jnp.float32),
            pltpu.VMEM((l, d2), jnp.float32),
            pltpu.VMEM((8, d2), jnp.float32),
            pltpu.VMEM((8, d2), jnp.float32),
            pltpu.SemaphoreType.DMA,
            pltpu.SemaphoreType.DMA,
        ],
        compiler_params=pltpu.CompilerParams(needs_layout_passes=False),
    )
    def fused_k(idx_hbm, tab_hbm, pe_hbm, g_hbm, b_hbm, out_hbm,
                idx_v, rows_v, pack_v, pe_v, g_v, b_v, gsem, osem):
        wid = lax.axis_index("s") * _NC + lax.axis_index("c")
        base = wid * per_w
        pltpu.sync_copy(
            idx_hbm.at[pl.ds(pl.multiple_of(wid * n_chunks, 8), n_chunks)],
            idx_v)
        pltpu.sync_copy(pe_hbm, pe_v)
        pltpu.sync_copy(g_hbm, g_v)
        pltpu.sync_copy(b_hbm, b_v)
        gs = [g_v[0, pl.ds(k * 16, 16)] for k in range(nk)]
        bs = [b_v[0, pl.ds(k * 16, 16)] for k in range(nk)]

        pltpu.async_copy(tab_hbm.at[idx_v.at[0]], rows_v.at[0], gsem)

        def body(i, carry):
            sl = i % 2
            r0 = pl.multiple_of(base + i * _G, 128)
            pltpu.make_async_copy(
                tab_hbm.at[idx_v.at[i]], rows_v.at[sl], gsem).wait()

            @pl.when(i + 1 < n_chunks)
            def _prefetch():
                pltpu.async_copy(
                    tab_hbm.at[idx_v.at[i + 1]], rows_v.at[1 - sl], gsem)

            @pl.when(i >= 2)
            def _drain():
                pltpu.make_async_copy(
                    pack_v.at[sl], out_hbm.at[pl.ds(0, _G)], osem).wait()

            pm = lax.rem(r0, l)

            def rows(jj, c2):
                j0 = jj * _U
                for u in range(_U):
                    j = j0 + u
                    p0 = pm + j
                    p = jnp.where(p0 >= l, p0 - l, p0)
                    xs = [rows_v[sl, j, pl.ds(k * 16, 16)]
                          + pe_v[p, pl.ds(k * 16, 16)] for k in range(nk)]
                    s = (xs[0] + xs[1]) + (xs[2] + xs[3])
                    q = (xs[0] * xs[0] + xs[1] * xs[1]
                         + xs[2] * xs[2] + xs[3] * xs[3])
                    total = jnp.sum(s)
                    sumsq = jnp.sum(q)
                    mean = total * (1.0 / d)
                    var = sumsq * (1.0 / d) - mean * mean
                    vv = jnp.full((16,), var + _EPS, dtype=jnp.float32)
                    rstd = _rsqrt_vec(vv)
                    for k in range(nk):
                        pack_v[sl, j, pl.ds(k * 16, 16)] = (
                            (xs[k] - mean) * rstd * gs[k] + bs[k])
                return c2

            lax.fori_loop(0, _G // _U, rows, 0)
            pltpu.async_copy(pack_v.at[sl], out_hbm.at[pl.ds(r0, _G)], osem)
            return carry

        lax.fori_loop(0, n_chunks, body, 0)
        pltpu.make_async_copy(
            pack_v.at[0], out_hbm.at[pl.ds(0, _G)], osem).wait()
        pltpu.make_async_copy(
            pack_v.at[1], out_hbm.at[pl.ds(0, _G)], osem).wait()

    return fused_k


def kernel(input_ids, W_emb, pe, ln_gamma, ln_beta):
    b, l = input_ids.shape
    v, d = W_emb.shape
    n = b * l
    idx = input_ids.reshape(n // _G, _G)
    tab = jnp.pad(W_emb, ((0, 0), (0, d)))
    pe_pad = jnp.pad(pe[:l], ((0, 0), (0, d)))
    g_pad = jnp.pad(ln_gamma.reshape(1, d), ((0, 7), (0, d)))
    b_pad = jnp.pad(ln_beta.reshape(1, d), ((0, 7), (0, d)))
    fused = _make_fused(n, d, l)
    out = fused(idx, tab, pe_pad, g_pad, b_pad)
    return out.reshape(b, l, d)
